# baseline (speedup 1.0000x reference)
import functools

import jax
import jax.numpy as jnp
from jax import lax
from jax.experimental import pallas as pl
from jax.experimental.pallas import tpu as pltpu
from jax.experimental.pallas import tpu_sc as plsc

_B, _S, _D = 4, 8192, 1024
_NG = 4
_NCAND = 32
_NL = _NG * _NCAND
_BLK = 4096
_SB = _S // _BLK


_SUB = 512


def _gate_kernel(x_ref, imp_ref, w_ref, g_ref, out_ref):
    s = pl.program_id(1)
    part = jnp.zeros((1, _NL), jnp.float32)
    for t in range(_BLK // _SUB):
        xt = x_ref[0, pl.ds(t * _SUB, _SUB), :]
        logits = jnp.dot(xt, w_ref[...], preferred_element_type=jnp.float32)
        e = jnp.exp(logits)
        gs = jnp.dot(e, g_ref[...], preferred_element_type=jnp.float32)
        p = e / gs
        imp_t = imp_ref[0, 0, :, pl.ds(t * _SUB, _SUB)]
        part = part + jnp.dot(imp_t, p, preferred_element_type=jnp.float32)

    @pl.when(s == 0)
    def _init():
        out_ref[0] = part

    @pl.when(s > 0)
    def _acc():
        out_ref[0] = out_ref[0] + part


def _gate_sums(x, importance, w_cat, group):
    return pl.pallas_call(
        _gate_kernel,
        grid=(_B, _SB),
        in_specs=[
            pl.BlockSpec((1, _BLK, _D), lambda b, s: (b, s, 0)),
            pl.BlockSpec((1, 1, 1, _BLK), lambda b, s: (b, s, 0, 0)),
            pl.BlockSpec((_D, _NL), lambda b, s: (0, 0)),
            pl.BlockSpec((_NL, _NL), lambda b, s: (0, 0)),
        ],
        out_specs=pl.BlockSpec((1, 1, _NL), lambda b, s: (b, 0, 0)),
        out_shape=jax.ShapeDtypeStruct((_B, 1, _NL), jnp.float32),
        compiler_params=pltpu.CompilerParams(
            dimension_semantics=("parallel", "arbitrary")),
    )(x, importance.reshape(_B, _SB, 1, _BLK), w_cat, group)


def _sc_topk_body(w_hbm, tw_hbm, ti_hbm, mask_hbm, w_v, tw_v, ti_v, mask_v):
    wid = lax.axis_index("s") * 2 + lax.axis_index("c")

    @pl.when(wid < _B * _NG)
    def _():
        b = wid // _NG
        r = wid % _NG
        pltpu.sync_copy(w_hbm.at[wid], w_v)
        v0 = w_v[pl.ds(0, 16)]
        v1 = w_v[pl.ds(16, 16)]
        total = jnp.sum(v0, axis=0) + jnp.sum(v1, axis=0)
        den = jnp.full((16,), total, jnp.float32) + 1e-8
        lane = lax.iota(jnp.int32, 16)
        s0, si0 = plsc.sort_key_val(v0 / den, lane, descending=True)
        s1, si1 = plsc.sort_key_val(v1 / den, lane + 16, descending=True)
        s1r = lax.rev(s1, (0,))
        si1r = lax.rev(si1, (0,))
        sel = s0 >= s1r
        tv, ti = plsc.sort_key_val(
            jnp.where(sel, s0, s1r), jnp.where(sel, si0, si1r),
            descending=True)
        k = jnp.where(r == 0, 8, 4)
        kmask = lane < k
        kf = kmask.astype(jnp.float32)
        ts = jnp.sum(tv * kf, axis=0)
        tden = jnp.full((16,), ts, jnp.float32) + 1e-8
        tw_v[...] = kf * (tv / tden)
        ti_v[...] = ti * kmask.astype(jnp.int32)
        pltpu.sync_copy(tw_v, tw_hbm.at[r, b])
        pltpu.sync_copy(ti_v, ti_hbm.at[r, b])
        zeros = jnp.zeros((16,), jnp.float32)
        mask_v[pl.ds(0, 16)] = zeros
        mask_v[pl.ds(16, 16)] = zeros
        plsc.store_scatter(mask_v, [ti], jnp.ones((16,), jnp.float32),
                           mask=kmask)
        pltpu.sync_copy(mask_v, mask_hbm.at[r, b])


def _sc_topk(w_flat):
    mesh = plsc.VectorSubcoreMesh(core_axis_name="c", subcore_axis_name="s")
    fn = pl.kernel(
        _sc_topk_body,
        out_type=[
            jax.ShapeDtypeStruct((_NG, _B, 16), jnp.float32),
            jax.ShapeDtypeStruct((_NG, _B, 16), jnp.int32),
            jax.ShapeDtypeStruct((_NG, _B, _NCAND), jnp.float32),
        ],
        mesh=mesh,
        scratch_types=[
            pltpu.VMEM((_NCAND,), jnp.float32),
            pltpu.VMEM((16,), jnp.float32),
            pltpu.VMEM((16,), jnp.int32),
            pltpu.VMEM((_NCAND,), jnp.float32),
        ],
        compiler_params=pltpu.CompilerParams(needs_layout_passes=False),
    )
    return fn(w_flat)


def kernel(x, importance, Wc, Wq, Wk, Wv):
    w_cat = jnp.concatenate([Wc, Wq, Wk, Wv], axis=1)
    group = jnp.kron(jnp.eye(_NG, dtype=jnp.float32),
                     jnp.ones((_NCAND, _NCAND), jnp.float32))
    w_all = _gate_sums(x, importance, w_cat, group)
    tw, ti, mask = _sc_topk(w_all.reshape(_B * _NG, _NCAND))
    return (tw[0, :, :8], ti[0, :, :8],
            tw[1, :, :4], ti[1, :, :4],
            tw[2, :, :4], ti[2, :, :4],
            tw[3, :, :4], ti[3, :, :4],
            mask[0], mask[1], mask[2], mask[3])


# speedup vs baseline: 1.1168x; 1.1168x over previous
import functools

import jax
import jax.numpy as jnp
from jax import lax
from jax.experimental import pallas as pl
from jax.experimental.pallas import tpu as pltpu
from jax.experimental.pallas import tpu_sc as plsc

_B, _S, _D = 4, 8192, 1024
_NG = 4
_NCAND = 32
_NL = _NG * _NCAND
_BLK = 4096
_SB = _S // _BLK


def _gate_kernel(x_ref, imp_ref, w_ref, g_ref, out_ref):
    s = pl.program_id(1)
    x = x_ref[0]
    logits = jnp.dot(x, w_ref[...], preferred_element_type=jnp.float32)
    e = jnp.exp(logits)
    eb = e.astype(jnp.bfloat16)
    gs = jnp.dot(eb, g_ref[...], preferred_element_type=jnp.float32)
    p = (e / gs).astype(jnp.bfloat16)
    part = jnp.dot(imp_ref[0, 0], p, preferred_element_type=jnp.float32)

    @pl.when(s == 0)
    def _init():
        out_ref[0] = part

    @pl.when(s > 0)
    def _acc():
        out_ref[0] = out_ref[0] + part


def _gate_sums(x, importance, w_cat, group):
    return pl.pallas_call(
        _gate_kernel,
        grid=(_B, _SB),
        in_specs=[
            pl.BlockSpec((1, _BLK, _D), lambda b, s: (b, s, 0)),
            pl.BlockSpec((1, 1, 1, _BLK), lambda b, s: (b, s, 0, 0)),
            pl.BlockSpec((_D, _NL), lambda b, s: (0, 0)),
            pl.BlockSpec((_NL, _NL), lambda b, s: (0, 0)),
        ],
        out_specs=pl.BlockSpec((1, 1, _NL), lambda b, s: (b, 0, 0)),
        out_shape=jax.ShapeDtypeStruct((_B, 1, _NL), jnp.float32),
        compiler_params=pltpu.CompilerParams(
            dimension_semantics=("parallel", "arbitrary")),
    )(x, importance.astype(jnp.bfloat16).reshape(_B, _SB, 1, _BLK),
      w_cat, group.astype(jnp.bfloat16))


def _sc_topk_body(w_hbm, tw_hbm, ti_hbm, mask_hbm, w_v, tw_v, ti_v, mask_v):
    wid = lax.axis_index("s") * 2 + lax.axis_index("c")

    @pl.when(wid < _B * _NG)
    def _():
        b = wid // _NG
        r = wid % _NG
        pltpu.sync_copy(w_hbm.at[wid], w_v)
        v0 = w_v[pl.ds(0, 16)]
        v1 = w_v[pl.ds(16, 16)]
        total = jnp.sum(v0, axis=0) + jnp.sum(v1, axis=0)
        den = jnp.full((16,), total, jnp.float32) + 1e-8
        lane = lax.iota(jnp.int32, 16)
        s0, si0 = plsc.sort_key_val(v0 / den, lane, descending=True)
        s1, si1 = plsc.sort_key_val(v1 / den, lane + 16, descending=True)
        s1r = lax.rev(s1, (0,))
        si1r = lax.rev(si1, (0,))
        sel = s0 >= s1r
        tv, ti = plsc.sort_key_val(
            jnp.where(sel, s0, s1r), jnp.where(sel, si0, si1r),
            descending=True)
        k = jnp.where(r == 0, 8, 4)
        kmask = lane < k
        kf = kmask.astype(jnp.float32)
        ts = jnp.sum(tv * kf, axis=0)
        tden = jnp.full((16,), ts, jnp.float32) + 1e-8
        tw_v[...] = kf * (tv / tden)
        ti_v[...] = ti * kmask.astype(jnp.int32)
        pltpu.sync_copy(tw_v, tw_hbm.at[r, b])
        pltpu.sync_copy(ti_v, ti_hbm.at[r, b])
        zeros = jnp.zeros((16,), jnp.float32)
        mask_v[pl.ds(0, 16)] = zeros
        mask_v[pl.ds(16, 16)] = zeros
        plsc.store_scatter(mask_v, [ti], jnp.ones((16,), jnp.float32),
                           mask=kmask)
        pltpu.sync_copy(mask_v, mask_hbm.at[r, b])


def _sc_topk(w_flat):
    mesh = plsc.VectorSubcoreMesh(core_axis_name="c", subcore_axis_name="s")
    fn = pl.kernel(
        _sc_topk_body,
        out_type=[
            jax.ShapeDtypeStruct((_NG, _B, 16), jnp.float32),
            jax.ShapeDtypeStruct((_NG, _B, 16), jnp.int32),
            jax.ShapeDtypeStruct((_NG, _B, _NCAND), jnp.float32),
        ],
        mesh=mesh,
        scratch_types=[
            pltpu.VMEM((_NCAND,), jnp.float32),
            pltpu.VMEM((16,), jnp.float32),
            pltpu.VMEM((16,), jnp.int32),
            pltpu.VMEM((_NCAND,), jnp.float32),
        ],
        compiler_params=pltpu.CompilerParams(needs_layout_passes=False),
    )
    return fn(w_flat)


def kernel(x, importance, Wc, Wq, Wk, Wv):
    w_cat = jnp.concatenate([Wc, Wq, Wk, Wv], axis=1)
    group = jnp.kron(jnp.eye(_NG, dtype=jnp.float32),
                     jnp.ones((_NCAND, _NCAND), jnp.float32))
    w_all = _gate_sums(x, importance, w_cat, group)
    tw, ti, mask = _sc_topk(w_all.reshape(_B * _NG, _NCAND))
    return (tw[0, :, :8], ti[0, :, :8],
            tw[1, :, :4], ti[1, :, :4],
            tw[2, :, :4], ti[2, :, :4],
            tw[3, :, :4], ti[3, :, :4],
            mask[0], mask[1], mask[2], mask[3])


# speedup vs baseline: 1.1177x; 1.0008x over previous
import functools

import jax
import jax.numpy as jnp
from jax import lax
from jax.experimental import pallas as pl
from jax.experimental.pallas import tpu as pltpu
from jax.experimental.pallas import tpu_sc as plsc

_B, _S, _D = 4, 8192, 1024
_NG = 4
_NCAND = 32
_NL = _NG * _NCAND
_BLK = 4096
_SB = _S // _BLK


def _gate_kernel(x_ref, imp_ref, w_ref, g_ref, out_ref):
    s = pl.program_id(1)
    x = x_ref[0]
    logits = jnp.dot(x, w_ref[...], preferred_element_type=jnp.float32)
    e = jnp.exp(logits)
    gs = jnp.dot(e, g_ref[...], preferred_element_type=jnp.float32)
    p = e / gs
    part = jnp.dot(imp_ref[0, 0], p, preferred_element_type=jnp.float32)

    @pl.when(s == 0)
    def _init():
        out_ref[0] = part

    @pl.when(s > 0)
    def _acc():
        out_ref[0] = out_ref[0] + part


def _gate_sums(x, importance, w_cat, group):
    return pl.pallas_call(
        _gate_kernel,
        grid=(_B, _SB),
        in_specs=[
            pl.BlockSpec((1, _BLK, _D), lambda b, s: (b, s, 0)),
            pl.BlockSpec((1, 1, 1, _BLK), lambda b, s: (b, s, 0, 0)),
            pl.BlockSpec((_D, _NL), lambda b, s: (0, 0)),
            pl.BlockSpec((_NL, _NL), lambda b, s: (0, 0)),
        ],
        out_specs=pl.BlockSpec((1, 1, _NL), lambda b, s: (b, 0, 0)),
        out_shape=jax.ShapeDtypeStruct((_B, 1, _NL), jnp.float32),
        compiler_params=pltpu.CompilerParams(
            dimension_semantics=("parallel", "arbitrary")),
    )(x, importance.reshape(_B, _SB, 1, _BLK), w_cat, group)


def _sc_topk_body(w_hbm, tw_hbm, ti_hbm, mask_hbm, w_v, tw_v, ti_v, mask_v):
    wid = lax.axis_index("s") * 2 + lax.axis_index("c")

    @pl.when(wid < _B * _NG)
    def _():
        b = wid // _NG
        r = wid % _NG
        pltpu.sync_copy(w_hbm.at[wid], w_v)
        v0 = w_v[pl.ds(0, 16)]
        v1 = w_v[pl.ds(16, 16)]
        total = jnp.sum(v0, axis=0) + jnp.sum(v1, axis=0)
        den = jnp.full((16,), total, jnp.float32) + 1e-8
        lane = lax.iota(jnp.int32, 16)
        s0, si0 = plsc.sort_key_val(v0 / den, lane, descending=True)
        s1, si1 = plsc.sort_key_val(v1 / den, lane + 16, descending=True)
        s1r = lax.rev(s1, (0,))
        si1r = lax.rev(si1, (0,))
        sel = s0 >= s1r
        tv, ti = plsc.sort_key_val(
            jnp.where(sel, s0, s1r), jnp.where(sel, si0, si1r),
            descending=True)
        k = jnp.where(r == 0, 8, 4)
        kmask = lane < k
        kf = kmask.astype(jnp.float32)
        ts = jnp.sum(tv * kf, axis=0)
        tden = jnp.full((16,), ts, jnp.float32) + 1e-8
        tw_v[...] = kf * (tv / tden)
        ti_v[...] = ti * kmask.astype(jnp.int32)
        pltpu.sync_copy(tw_v, tw_hbm.at[r, b])
        pltpu.sync_copy(ti_v, ti_hbm.at[r, b])
        zeros = jnp.zeros((16,), jnp.float32)
        mask_v[pl.ds(0, 16)] = zeros
        mask_v[pl.ds(16, 16)] = zeros
        plsc.store_scatter(mask_v, [ti], jnp.ones((16,), jnp.float32),
                           mask=kmask)
        pltpu.sync_copy(mask_v, mask_hbm.at[r, b])


def _sc_topk(w_flat):
    mesh = plsc.VectorSubcoreMesh(core_axis_name="c", subcore_axis_name="s")
    fn = pl.kernel(
        _sc_topk_body,
        out_type=[
            jax.ShapeDtypeStruct((_NG, _B, 16), jnp.float32),
            jax.ShapeDtypeStruct((_NG, _B, 16), jnp.int32),
            jax.ShapeDtypeStruct((_NG, _B, _NCAND), jnp.float32),
        ],
        mesh=mesh,
        scratch_types=[
            pltpu.VMEM((_NCAND,), jnp.float32),
            pltpu.VMEM((16,), jnp.float32),
            pltpu.VMEM((16,), jnp.int32),
            pltpu.VMEM((_NCAND,), jnp.float32),
        ],
        compiler_params=pltpu.CompilerParams(needs_layout_passes=False),
    )
    return fn(w_flat)


def kernel(x, importance, Wc, Wq, Wk, Wv):
    w_cat = jnp.concatenate([Wc, Wq, Wk, Wv], axis=1)
    group = jnp.kron(jnp.eye(_NG, dtype=jnp.float32),
                     jnp.ones((_NCAND, _NCAND), jnp.float32))
    w_all = _gate_sums(x, importance, w_cat, group)
    tw, ti, mask = _sc_topk(w_all.reshape(_B * _NG, _NCAND))
    return (tw[0, :, :8], ti[0, :, :8],
            tw[1, :, :4], ti[1, :, :4],
            tw[2, :, :4], ti[2, :, :4],
            tw[3, :, :4], ti[3, :, :4],
            mask[0], mask[1], mask[2], mask[3])


# speedup vs baseline: 1.1283x; 1.0095x over previous
import functools

import jax
import jax.numpy as jnp
from jax import lax
from jax.experimental import pallas as pl
from jax.experimental.pallas import tpu as pltpu
from jax.experimental.pallas import tpu_sc as plsc

_B, _S, _D = 4, 8192, 1024
_NG = 4
_NCAND = 32
_NL = _NG * _NCAND
_BLK = 4096
_SB = _S // _BLK


def _gate_kernel(x_ref, imp_ref, w_ref, g_ref, out_ref):
    s = pl.program_id(1)
    x = x_ref[0]
    logits = jnp.dot(x, w_ref[...], preferred_element_type=jnp.float32)
    e = jnp.exp(logits)
    gs = jnp.dot(e, g_ref[...], preferred_element_type=jnp.float32)
    p = e / gs
    part = jnp.dot(imp_ref[0, 0], p, preferred_element_type=jnp.float32)

    @pl.when(s == 0)
    def _init():
        out_ref[0] = part

    @pl.when(s > 0)
    def _acc():
        out_ref[0] = out_ref[0] + part


def _gate_sums(x, importance, w_cat, group):
    return pl.pallas_call(
        _gate_kernel,
        grid=(_B, _SB),
        in_specs=[
            pl.BlockSpec((1, _BLK, _D), lambda b, s: (b, s, 0)),
            pl.BlockSpec((1, 1, 1, _BLK), lambda b, s: (b, s, 0, 0)),
            pl.BlockSpec((_D, _NL), lambda b, s: (0, 0)),
            pl.BlockSpec((_NL, _NL), lambda b, s: (0, 0)),
        ],
        out_specs=pl.BlockSpec((1, 1, _NL), lambda b, s: (b, 0, 0)),
        out_shape=jax.ShapeDtypeStruct((_B, 1, _NL), jnp.float32),
        compiler_params=pltpu.CompilerParams(
            dimension_semantics=("parallel", "arbitrary")),
    )(x, importance.reshape(_B, _SB, 1, _BLK), w_cat, group)


def _sc_topk_body(w_hbm, out_hbm, w_v, out_v):
    wid = lax.axis_index("s") * 2 + lax.axis_index("c")

    @pl.when(wid < _B * _NG)
    def _():
        b = wid // _NG
        r = wid % _NG
        pltpu.sync_copy(w_hbm.at[wid], w_v)
        v0 = w_v[pl.ds(0, 16)]
        v1 = w_v[pl.ds(16, 16)]
        total = jnp.sum(v0, axis=0) + jnp.sum(v1, axis=0)
        den = jnp.full((16,), total, jnp.float32) + 1e-8
        lane = lax.iota(jnp.int32, 16)
        s0, si0 = plsc.sort_key_val(v0 / den, lane, descending=True)
        s1, si1 = plsc.sort_key_val(v1 / den, lane + 16, descending=True)
        s1r = lax.rev(s1, (0,))
        si1r = lax.rev(si1, (0,))
        sel = s0 >= s1r
        tv, ti = plsc.sort_key_val(
            jnp.where(sel, s0, s1r), jnp.where(sel, si0, si1r),
            descending=True)
        k = jnp.where(r == 0, 8, 4)
        kmask = lane < k
        kf = kmask.astype(jnp.float32)
        ts = jnp.sum(tv * kf, axis=0)
        tden = jnp.full((16,), ts, jnp.float32) + 1e-8
        out_v[pl.ds(0, 16)] = kf * (tv / tden)
        out_v[pl.ds(16, 16)] = plsc.bitcast(ti * kmask.astype(jnp.int32),
                                            jnp.float32)
        zeros = jnp.zeros((16,), jnp.float32)
        out_v[pl.ds(32, 16)] = zeros
        out_v[pl.ds(48, 16)] = zeros
        plsc.store_scatter(out_v, [ti + 32], jnp.ones((16,), jnp.float32),
                           mask=kmask)
        pltpu.sync_copy(out_v, out_hbm.at[r, b])


def _sc_topk(w_flat):
    mesh = plsc.VectorSubcoreMesh(core_axis_name="c", subcore_axis_name="s")
    fn = pl.kernel(
        _sc_topk_body,
        out_type=jax.ShapeDtypeStruct((_NG, _B, 64), jnp.float32),
        mesh=mesh,
        scratch_types=[
            pltpu.VMEM((_NCAND,), jnp.float32),
            pltpu.VMEM((64,), jnp.float32),
        ],
        compiler_params=pltpu.CompilerParams(needs_layout_passes=False),
    )
    return fn(w_flat)


def kernel(x, importance, Wc, Wq, Wk, Wv):
    w_cat = jnp.concatenate([Wc, Wq, Wk, Wv], axis=1)
    group = jnp.kron(jnp.eye(_NG, dtype=jnp.float32),
                     jnp.ones((_NCAND, _NCAND), jnp.float32))
    w_all = _gate_sums(x, importance, w_cat, group)
    packed = _sc_topk(w_all.reshape(_B * _NG, _NCAND))
    tw = packed[:, :, :16]
    ti = jax.lax.bitcast_convert_type(packed[:, :, 16:32], jnp.int32)
    mask = packed[:, :, 32:]
    return (tw[0, :, :8], ti[0, :, :8],
            tw[1, :, :4], ti[1, :, :4],
            tw[2, :, :4], ti[2, :, :4],
            tw[3, :, :4], ti[3, :, :4],
            mask[0], mask[1], mask[2], mask[3])
